# Initial kernel scaffold; baseline (speedup 1.0000x reference)
import jax
import jax.numpy as jnp
from jax.experimental import pallas as pl


def kernel(x, words, word_lengths):
    raise NotImplementedError("write your pallas kernel here")


# speedup vs baseline: 139.7448x; 139.7448x over previous
import functools

import jax
import jax.numpy as jnp
from jax import lax
from jax.experimental import pallas as pl
from jax.experimental.pallas import tpu as pltpu
from jax.experimental.pallas import tpu_sc as plsc

NUM_CHARS = 96
MAXW = 11
MAXL = MAXW + 1
BSZ, SEQ, NUM_WORDS = 4, 8, 1024
NBS = BSZ * SEQ
NWORKERS = 32
WPW = NUM_WORDS // NWORKERS
LANES = 16
GROUPS = WPW // LANES
NTASK = NBS * GROUPS
D = MAXW + 2

def _dl_body(x_hbm, wt_hbm, wl_hbm, out_hbm, x_v, wt_v, wl_v, swl_v, dtab,
             res_v):
    wid = lax.axis_index("s") * 2 + lax.axis_index("c")
    pltpu.sync_copy(x_hbm, x_v)
    pltpu.sync_copy(wt_hbm.at[wid], wt_v)
    pltpu.sync_copy(wl_hbm.at[wid], wl_v)
    lanes = lax.broadcasted_iota(jnp.int32, (LANES,), 0)
    zf = jnp.zeros((LANES,), jnp.float32)
    zi = jnp.zeros((LANES,), jnp.int32)

    for bs in range(NBS):
        xvecf = x_v[bs, :].astype(jnp.float32)
        m = jnp.max(xvecf)
        swl_v[bs, :] = plsc.all_reduce_ffs(xvecf == jnp.full((LANES,), m))

    def task_body(t, carry):
        bs = t // GROUPS
        goff = (t % GROUPS) * LANES
        xvec = x_v[bs, :]
        swl_vec = swl_v[bs, :]

        wl_vec = wl_v[pl.ds(goff, LANES)]
        wch = [wt_v[jj, pl.ds(goff, LANES)] for jj in range(MAXW)]
        maxdist = (wl_vec + swl_vec).astype(jnp.float32)

        for r in range(D):
            dtab[r * D, :] = maxdist
        for c in range(1, D):
            dtab[c, :] = maxdist
        for c in range(1, D):
            dtab[D + c, :] = jnp.where(c - 1 <= wl_vec, jnp.float32(c - 1), 0.0)
        for r in range(2, D):
            dtab[r * D + 1, :] = jnp.where(
                r - 1 <= swl_vec, jnp.float32(r - 1), 0.0)

        kf = [zf] * MAXL
        for i in range(1, MAXL):
            xcv = jnp.full((LANES,), xvec[i - 1])
            prev = jnp.where(i <= swl_vec, jnp.float32(i), 0.0)
            topleft = dtab[i * D + 1, :]
            dbi = zi
            dbf = zf
            for j in range(1, MAXL):
                top = dtab[i * D + j + 1, :]
                meq = wch[j - 1] == xcv
                cost = jnp.where(meq, 0.0, 1.0)
                ki = kf[j].astype(jnp.int32)
                dt = plsc.load_gather(dtab, [ki * D + dbi, lanes])
                c3 = topleft + cost
                c4 = dt - kf[j] - dbf + jnp.float32(i + j - 1)
                val = jnp.minimum(jnp.minimum(top, prev) + 1.0,
                                  jnp.minimum(c3, c4))
                dtab[(i + 1) * D + j + 1, :] = val
                kf[j] = jnp.where(meq, jnp.float32(i), kf[j])
                dbi = jnp.where(meq, jnp.int32(j), dbi)
                dbf = jnp.where(meq, jnp.float32(j), dbf)
                prev = val
                topleft = top

        outv = plsc.load_gather(
            dtab, [(swl_vec + 1) * D + wl_vec + 1, lanes])
        res_v[t, :] = outv
        return carry

    lax.fori_loop(0, NTASK, task_body, 0)
    pltpu.sync_copy(res_v, out_hbm.at[wid])


@functools.lru_cache(maxsize=1)
def _build():
    mesh = plsc.VectorSubcoreMesh(
        core_axis_name="c", subcore_axis_name="s", num_cores=2, num_subcores=16)
    return pl.kernel(
        _dl_body,
        out_type=jax.ShapeDtypeStruct((NWORKERS, NTASK, LANES), jnp.float32),
        mesh=mesh,
        scratch_types=[
            pltpu.VMEM((NBS, LANES), jnp.int32),
            pltpu.VMEM((MAXW, WPW), jnp.int32),
            pltpu.VMEM((WPW,), jnp.int32),
            pltpu.VMEM((NBS, LANES), jnp.int32),
            pltpu.VMEM((D * D, LANES), jnp.float32),
            pltpu.VMEM((NTASK, LANES), jnp.float32),
        ],
        compiler_params=pltpu.CompilerParams(needs_layout_passes=False),
    )


def kernel(x, words, word_lengths):
    xf = x.reshape(NBS, MAXL)
    xf = jnp.pad(xf, ((0, 0), (0, LANES - MAXL)), constant_values=-1)
    wt = words.T.reshape(MAXW, NWORKERS, WPW).transpose(1, 0, 2)
    wlc = word_lengths.reshape(NWORKERS, WPW)
    out = _build()(xf, wt, wlc)
    out = out.reshape(NWORKERS, NBS, GROUPS, LANES)
    out = out.transpose(1, 0, 2, 3).reshape(BSZ, SEQ, NUM_WORDS)
    return out


# speedup vs baseline: 306.5391x; 2.1936x over previous
import functools

import jax
import jax.numpy as jnp
from jax import lax
from jax.experimental import pallas as pl
from jax.experimental.pallas import tpu as pltpu
from jax.experimental.pallas import tpu_sc as plsc

NUM_CHARS = 96
MAXW = 11
MAXL = MAXW + 1
BSZ, SEQ, NUM_WORDS = 4, 8, 1024
NBS = BSZ * SEQ
NWORKERS = 32
WPW = NUM_WORDS // NWORKERS
LANES = 16
GROUPS = WPW // LANES
NTASK = NBS * GROUPS
D = MAXW + 2

def _dl_body(x_hbm, wt_hbm, wl_hbm, out_hbm, x_v, wt_v, wl_v, swl_v, dtab,
             res_v):
    wid = lax.axis_index("s") * 2 + lax.axis_index("c")
    pltpu.sync_copy(x_hbm, x_v)
    pltpu.sync_copy(wt_hbm.at[wid], wt_v)
    pltpu.sync_copy(wl_hbm.at[wid], wl_v)
    lanes = lax.broadcasted_iota(jnp.int32, (LANES,), 0)
    zi = jnp.zeros((LANES,), jnp.int32)

    for bs in range(NBS):
        xvecf = x_v[bs, :].astype(jnp.float32)
        m = jnp.max(xvecf)
        swl_v[bs, :] = plsc.all_reduce_ffs(xvecf == jnp.full((LANES,), m))

    def task_body(t, carry):
        bs = t // GROUPS
        goff = (t % GROUPS) * LANES
        xvec = x_v[bs, :]
        swl_vec = swl_v[bs, :]

        wl_vec = wl_v[pl.ds(goff, LANES)]
        wch = [wt_v[jj, pl.ds(goff, LANES)] for jj in range(MAXW)]
        maxdist = (wl_vec + swl_vec).astype(jnp.float32)

        for r in range(D):
            dtab[r * D, :] = maxdist - jnp.float32(r)
        for c in range(1, D):
            dtab[c, :] = maxdist - jnp.float32(c)
        for c in range(1, D):
            dtab[D + c, :] = jnp.where(
                c - 1 <= wl_vec, jnp.float32(-2), jnp.float32(-(c + 1)))
        for r in range(2, D):
            dtab[r * D + 1, :] = jnp.where(
                r - 1 <= swl_vec, jnp.float32(-2), jnp.float32(-(r + 1)))

        kd = [zi] * MAXL
        for i in range(1, MAXL):
            xcv = jnp.full((LANES,), xvec[i - 1])
            prev = jnp.where(i <= swl_vec, jnp.float32(-2), jnp.float32(-(i + 2)))
            topleft = dtab[i * D + 1, :]
            dbi = zi
            for j in range(1, MAXL):
                top = dtab[i * D + j + 1, :]
                meq = wch[j - 1] == xcv
                dt = plsc.load_gather(dtab, [kd[j] + dbi, lanes])
                c3 = topleft + jnp.where(meq, jnp.float32(-2), jnp.float32(-1))
                val = jnp.minimum(jnp.minimum(top, prev),
                                  jnp.minimum(c3, dt - 3.0))
                dtab[(i + 1) * D + j + 1, :] = val
                kd[j] = jnp.where(meq, jnp.int32(i * D), kd[j])
                dbi = jnp.where(meq, jnp.int32(j), dbi)
                prev = val
                topleft = top

        outv = plsc.load_gather(
            dtab, [(swl_vec + 1) * D + wl_vec + 1, lanes])
        res_v[t, :] = outv + maxdist + 2.0
        return carry

    lax.fori_loop(0, NTASK, task_body, 0)
    pltpu.sync_copy(res_v, out_hbm.at[wid])


@functools.lru_cache(maxsize=1)
def _build():
    mesh = plsc.VectorSubcoreMesh(
        core_axis_name="c", subcore_axis_name="s", num_cores=2, num_subcores=16)
    return pl.kernel(
        _dl_body,
        out_type=jax.ShapeDtypeStruct((NWORKERS, NTASK, LANES), jnp.float32),
        mesh=mesh,
        scratch_types=[
            pltpu.VMEM((NBS, LANES), jnp.int32),
            pltpu.VMEM((MAXW, WPW), jnp.int32),
            pltpu.VMEM((WPW,), jnp.int32),
            pltpu.VMEM((NBS, LANES), jnp.int32),
            pltpu.VMEM((D * D, LANES), jnp.float32),
            pltpu.VMEM((NTASK, LANES), jnp.float32),
        ],
        compiler_params=pltpu.CompilerParams(needs_layout_passes=False),
    )


def kernel(x, words, word_lengths):
    xf = x.reshape(NBS, MAXL)
    xf = jnp.pad(xf, ((0, 0), (0, LANES - MAXL)), constant_values=-1)
    wt = words.T.reshape(MAXW, NWORKERS, WPW).transpose(1, 0, 2)
    wlc = word_lengths.reshape(NWORKERS, WPW)
    out = _build()(xf, wt, wlc)
    out = out.reshape(NWORKERS, NBS, GROUPS, LANES)
    out = out.transpose(1, 0, 2, 3).reshape(BSZ, SEQ, NUM_WORDS)
    return out


# speedup vs baseline: 413.3811x; 1.3485x over previous
import functools

import jax
import jax.numpy as jnp
from jax import lax
from jax.experimental import pallas as pl
from jax.experimental.pallas import tpu as pltpu
from jax.experimental.pallas import tpu_sc as plsc

NUM_CHARS = 96
MAXW = 11
MAXL = MAXW + 1
BSZ, SEQ, NUM_WORDS = 4, 8, 1024
NBS = BSZ * SEQ
NWORKERS = 32
WPW = NUM_WORDS // NWORKERS
LANES = 16
GROUPS = WPW // LANES
NTASK = NBS * GROUPS
D = MAXW + 2

def _dl_body(x_hbm, wt_hbm, wl_hbm, out_hbm, x_v, wt_v, wl_v, swl_v, dtab,
             res_v):
    wid = lax.axis_index("s") * 2 + lax.axis_index("c")
    pltpu.sync_copy(x_hbm, x_v)
    pltpu.sync_copy(wt_hbm.at[wid], wt_v)
    pltpu.sync_copy(wl_hbm.at[wid], wl_v)
    lanes = lax.broadcasted_iota(jnp.int32, (LANES,), 0)
    zi = jnp.zeros((LANES,), jnp.int32)

    for bs in range(NBS):
        xvecf = x_v[bs, :].astype(jnp.float32)
        m = jnp.max(xvecf)
        swl_v[bs, :] = plsc.all_reduce_ffs(xvecf == jnp.full((LANES,), m))

    def task_body(t, carry):
        bs = t // GROUPS
        goff = (t % GROUPS) * LANES
        xvec = x_v[bs, :]
        swl_vec = swl_v[bs, :]

        wl_vec = wl_v[pl.ds(goff, LANES)]
        wch = [wt_v[jj, pl.ds(goff, LANES)] for jj in range(MAXW)]
        maxdist = (wl_vec + swl_vec).astype(jnp.float32)

        for r in range(D):
            dtab[r * D, :] = maxdist - jnp.float32(r)
        for c in range(1, D):
            dtab[c, :] = maxdist - jnp.float32(c)
        for c in range(1, D):
            dtab[D + c, :] = jnp.where(
                c - 1 <= wl_vec, jnp.float32(-2), jnp.float32(-(c + 1)))
        for r in range(2, D):
            dtab[r * D + 1, :] = jnp.where(
                r - 1 <= swl_vec, jnp.float32(-2), jnp.float32(-(r + 1)))

        swl_s = swl_vec[0]
        bsv = jnp.full((LANES,), bs)

        def row_body(i, kd):
            kd = list(kd)
            xcv = plsc.load_gather(x_v, [bsv, jnp.full((LANES,), i - 1)])
            fi = jnp.full((LANES,), i).astype(jnp.float32)
            prev = jnp.where(i <= swl_vec, jnp.float32(-2), -(fi + 2.0))
            row = i * D
            idv = jnp.full((LANES,), row)
            topleft = dtab[row + 1, :]
            dbi = zi
            for j in range(1, MAXL):
                top = dtab[row + j + 1, :]
                meq = wch[j - 1] == xcv
                dt = plsc.load_gather(dtab, [kd[j] + dbi, lanes])
                c3 = topleft + jnp.where(meq, jnp.float32(-2), jnp.float32(-1))
                val = jnp.minimum(jnp.minimum(top, prev),
                                  jnp.minimum(c3, dt - 3.0))
                dtab[row + D + j + 1, :] = val
                kd[j] = jnp.where(meq, idv, kd[j])
                dbi = jnp.where(meq, jnp.int32(j), dbi)
                prev = val
                topleft = top
            return tuple(kd)

        lax.fori_loop(1, swl_s + 1, row_body, tuple([zi] * MAXL))

        outv = plsc.load_gather(
            dtab, [(swl_vec + 1) * D + wl_vec + 1, lanes])
        res_v[t, :] = outv + maxdist + 2.0
        return carry

    lax.fori_loop(0, NTASK, task_body, 0)
    pltpu.sync_copy(res_v, out_hbm.at[wid])


@functools.lru_cache(maxsize=1)
def _build():
    mesh = plsc.VectorSubcoreMesh(
        core_axis_name="c", subcore_axis_name="s", num_cores=2, num_subcores=16)
    return pl.kernel(
        _dl_body,
        out_type=jax.ShapeDtypeStruct((NWORKERS, NTASK, LANES), jnp.float32),
        mesh=mesh,
        scratch_types=[
            pltpu.VMEM((NBS, LANES), jnp.int32),
            pltpu.VMEM((MAXW, WPW), jnp.int32),
            pltpu.VMEM((WPW,), jnp.int32),
            pltpu.VMEM((NBS, LANES), jnp.int32),
            pltpu.VMEM((D * D, LANES), jnp.float32),
            pltpu.VMEM((NTASK, LANES), jnp.float32),
        ],
        compiler_params=pltpu.CompilerParams(needs_layout_passes=False),
    )


def kernel(x, words, word_lengths):
    xf = x.reshape(NBS, MAXL)
    xf = jnp.pad(xf, ((0, 0), (0, LANES - MAXL)), constant_values=-1)
    wt = words.T.reshape(MAXW, NWORKERS, WPW).transpose(1, 0, 2)
    wlc = word_lengths.reshape(NWORKERS, WPW)
    out = _build()(xf, wt, wlc)
    out = out.reshape(NWORKERS, NBS, GROUPS, LANES)
    out = out.transpose(1, 0, 2, 3).reshape(BSZ, SEQ, NUM_WORDS)
    return out
